# baseline (speedup 1.0000x reference)
import functools

import jax
import jax.numpy as jnp
from jax import lax
from jax.experimental import pallas as pl
from jax.experimental.pallas import tpu as pltpu
from jax.experimental.pallas import tpu_sc as plsc

_D = 1280
_LANES = 16


def _build_lookup(num_tokens: int, chunk: int):
    info = plsc.get_sparse_core_info()
    nc, ns = info.num_cores, info.num_subcores
    nw = nc * ns
    tpw = num_tokens // nw
    assert num_tokens % nw == 0 and tpw % chunk == 0 and tpw % _LANES == 0
    nchunks = tpw // chunk

    mesh = plsc.VectorSubcoreMesh(core_axis_name="c", subcore_axis_name="s")

    @functools.partial(
        pl.kernel,
        mesh=mesh,
        out_type=jax.ShapeDtypeStruct((num_tokens, 2, _D), jnp.float32),
        scratch_types=[
            pltpu.VMEM((tpw,), jnp.int32),
            pltpu.VMEM((tpw,), jnp.int32),
            pltpu.VMEM((2, chunk, _D), jnp.float32),
            pltpu.VMEM((2, chunk, _D), jnp.float32),
            pltpu.SemaphoreType.DMA,
            pltpu.SemaphoreType.DMA,
            pltpu.SemaphoreType.DMA,
            pltpu.SemaphoreType.DMA,
        ],
    )
    def lookup(x_hbm, c2a_hbm, codon_hbm, aa_hbm, out_hbm,
               x_v, aa_v, cbuf, abuf, gsem, asem, wsem_c, wsem_a):
        wid = lax.axis_index("s") * nc + lax.axis_index("c")
        base = wid * tpw

        pltpu.sync_copy(x_hbm.at[pl.ds(base, tpw)], x_v)

        for i in range(tpw // 128):
            pltpu.async_copy(
                c2a_hbm.at[x_v.at[pl.ds(i * 128, 128)]],
                aa_v.at[pl.ds(i * 128, 128)], gsem).wait()

        def fire(c, slot):
            off = c * chunk
            g = pltpu.async_copy(
                codon_hbm.at[x_v.at[pl.ds(off, chunk)]], cbuf.at[slot], gsem)
            a = pltpu.async_copy(
                aa_hbm.at[aa_v.at[pl.ds(off, chunk)]], abuf.at[slot], asem)
            return g, a

        def drain(c, slot):
            off = c * chunk
            wc = pltpu.async_copy(
                cbuf.at[slot], out_hbm.at[pl.ds(base + off, chunk), 0], wsem_c)
            wa = pltpu.async_copy(
                abuf.at[slot], out_hbm.at[pl.ds(base + off, chunk), 1], wsem_a)
            return wc, wa

        g, a = fire(0, 0)
        writes = (None, None)
        for c in range(nchunks):
            slot = c % 2
            g.wait()
            a.wait()
            if c + 1 < nchunks:
                if writes[0] is not None:
                    writes[0].wait()
                    writes[1].wait()
                g, a = fire(c + 1, 1 - slot)
            writes = drain(c, slot)
        writes[0].wait()
        writes[1].wait()

    return lookup


def kernel(x, aa_table, codon_table, codon_to_aa):
    b, l = x.shape
    n = b * l
    xf = x.reshape(n).astype(jnp.int32)
    c2a = jnp.pad(codon_to_aa.astype(jnp.int32), (0, 80 - codon_to_aa.shape[0]))
    lookup = _build_lookup(n, 16)
    out = lookup(xf, c2a, codon_table, aa_table)
    return out.reshape(b, l, 2 * _D)


# speedup vs baseline: 2.4412x; 2.4412x over previous
import functools

import jax
import jax.numpy as jnp
from jax import lax
from jax.experimental import pallas as pl
from jax.experimental.pallas import tpu as pltpu
from jax.experimental.pallas import tpu_sc as plsc

_D = 1280


def _build_lookup(batch: int, seqlen: int, chunk: int):
    info = plsc.get_sparse_core_info()
    nc, ns = info.num_cores, info.num_subcores
    nw = nc * ns
    num_tokens = batch * seqlen
    tpw = num_tokens // nw
    assert num_tokens % nw == 0 and tpw % chunk == 0 and seqlen % tpw == 0
    nchunks = tpw // chunk
    wpb = seqlen // tpw

    mesh = plsc.VectorSubcoreMesh(core_axis_name="c", subcore_axis_name="s")

    @functools.partial(
        pl.kernel,
        mesh=mesh,
        out_type=jax.ShapeDtypeStruct((batch, seqlen, 2 * _D), jnp.float32),
        scratch_types=[
            pltpu.VMEM((tpw,), jnp.int32),
            pltpu.VMEM((tpw,), jnp.int32),
            pltpu.VMEM((2, chunk, _D), jnp.float32),
            pltpu.VMEM((2, chunk, _D), jnp.float32),
            pltpu.SemaphoreType.DMA,
            pltpu.SemaphoreType.DMA,
            pltpu.SemaphoreType.DMA,
            pltpu.SemaphoreType.DMA,
        ],
    )
    def lookup(x_hbm, c2a_hbm, codon_hbm, aa_hbm, out_hbm,
               x_v, aa_v, cbuf, abuf, gsem, asem, wsem_c, wsem_a):
        wid = lax.axis_index("s") * nc + lax.axis_index("c")
        b = wid // wpb
        l0 = (wid % wpb) * tpw

        pltpu.sync_copy(x_hbm.at[b, pl.ds(l0, tpw)], x_v)

        for i in range(tpw // 128):
            pltpu.async_copy(
                c2a_hbm.at[x_v.at[pl.ds(i * 128, 128)]],
                aa_v.at[pl.ds(i * 128, 128)], gsem).wait()

        def fire(c, slot):
            off = c * chunk
            g = pltpu.async_copy(
                codon_hbm.at[x_v.at[pl.ds(off, chunk)]], cbuf.at[slot], gsem)
            a = pltpu.async_copy(
                aa_hbm.at[aa_v.at[pl.ds(off, chunk)]], abuf.at[slot], asem)
            return g, a

        def drain(c, slot):
            off = c * chunk
            wc = pltpu.async_copy(
                cbuf.at[slot],
                out_hbm.at[b, pl.ds(l0 + off, chunk), pl.ds(0, _D)], wsem_c)
            wa = pltpu.async_copy(
                abuf.at[slot],
                out_hbm.at[b, pl.ds(l0 + off, chunk), pl.ds(_D, _D)], wsem_a)
            return wc, wa

        g, a = fire(0, 0)
        writes = (None, None)
        for c in range(nchunks):
            slot = c % 2
            g.wait()
            a.wait()
            if c + 1 < nchunks:
                if writes[0] is not None:
                    writes[0].wait()
                    writes[1].wait()
                g, a = fire(c + 1, 1 - slot)
            writes = drain(c, slot)
        writes[0].wait()
        writes[1].wait()

    return lookup


def kernel(x, aa_table, codon_table, codon_to_aa):
    b, l = x.shape
    xi = x.astype(jnp.int32)
    c2a = jnp.pad(codon_to_aa.astype(jnp.int32), (0, 80 - codon_to_aa.shape[0]))
    lookup = _build_lookup(b, l, 16)
    return lookup(xi, c2a, codon_table, aa_table)


# speedup vs baseline: 3.8304x; 1.5691x over previous
import functools

import jax
import jax.numpy as jnp
from jax import lax
from jax.experimental import pallas as pl
from jax.experimental.pallas import tpu as pltpu
from jax.experimental.pallas import tpu_sc as plsc

_D = 1280
_VPAD = 128


def _build_lookup(batch: int, seqlen: int, chunk: int):
    info = plsc.get_sparse_core_info()
    nc, ns = info.num_cores, info.num_subcores
    num_tokens = batch * seqlen
    tpw = num_tokens // (nc * ns)
    assert num_tokens % (nc * ns) == 0 and tpw % chunk == 0 and seqlen % tpw == 0
    nchunks = tpw // chunk
    wpb = seqlen // tpw
    rps = _VPAD // ns

    mesh = plsc.VectorSubcoreMesh(core_axis_name="c", subcore_axis_name="s")

    @functools.partial(
        pl.kernel,
        mesh=mesh,
        out_type=(
            jax.ShapeDtypeStruct((batch, seqlen, 2 * _D), jnp.float32),
            jax.ShapeDtypeStruct((nc, _VPAD, 2 * _D), jnp.float32),
        ),
        scratch_types=[
            pltpu.VMEM((tpw,), jnp.int32),
            pltpu.VMEM((rps,), jnp.int32),
            pltpu.VMEM((rps, _D), jnp.float32),
            pltpu.VMEM((2, chunk, 2 * _D), jnp.float32),
            pltpu.SemaphoreType.DMA,
            pltpu.SemaphoreType.DMA,
        ],
    )
    def lookup(x_hbm, c2a_hbm, codon_hbm, aa_hbm, out_hbm, fused_hbm,
               x_v, idx_v, row_v, buf, gsem, wsem):
        sc = lax.axis_index("c")
        sid = lax.axis_index("s")
        wid = sid * nc + sc
        b = wid // wpb
        l0 = (wid % wpb) * tpw
        r0 = sid * rps

        myfused = fused_hbm.at[sc]
        pltpu.sync_copy(c2a_hbm.at[pl.ds(r0, rps)], idx_v)
        pltpu.async_copy(aa_hbm.at[idx_v], row_v, gsem).wait()
        pltpu.sync_copy(row_v, myfused.at[pl.ds(r0, rps), pl.ds(_D, _D)])
        pltpu.sync_copy(codon_hbm.at[pl.ds(r0, rps)], row_v)
        pltpu.sync_copy(row_v, myfused.at[pl.ds(r0, rps), pl.ds(0, _D)])
        plsc.subcore_barrier()

        pltpu.sync_copy(x_hbm.at[b, pl.ds(l0, tpw)], x_v)

        def fire(c, slot):
            return pltpu.async_copy(
                myfused.at[x_v.at[pl.ds(c * chunk, chunk)]], buf.at[slot], gsem)

        def drain(c, slot):
            return pltpu.async_copy(
                buf.at[slot],
                out_hbm.at[b, pl.ds(l0 + c * chunk, chunk)], wsem)

        g = fire(0, 0)
        w = None
        for c in range(nchunks):
            slot = c % 2
            g.wait()
            if c + 1 < nchunks:
                if w is not None:
                    w.wait()
                g = fire(c + 1, 1 - slot)
            w = drain(c, slot)
        w.wait()

    return lookup


def kernel(x, aa_table, codon_table, codon_to_aa):
    b, l = x.shape
    v = codon_table.shape[0]
    xi = x.astype(jnp.int32)
    c2a = jnp.pad(codon_to_aa.astype(jnp.int32), (0, _VPAD - v))
    codon_p = jnp.pad(codon_table, ((0, _VPAD - v), (0, 0)))
    lookup = _build_lookup(b, l, 16)
    out, _ = lookup(xi, c2a, codon_p, aa_table)
    return out


# speedup vs baseline: 3.8597x; 1.0076x over previous
import functools

import jax
import jax.numpy as jnp
from jax import lax
from jax.experimental import pallas as pl
from jax.experimental.pallas import tpu as pltpu
from jax.experimental.pallas import tpu_sc as plsc

_D = 1280
_VPAD = 128
_NBUF = 3


def _build_lookup(batch: int, seqlen: int, chunk: int):
    info = plsc.get_sparse_core_info()
    nc, ns = info.num_cores, info.num_subcores
    num_tokens = batch * seqlen
    tpw = num_tokens // (nc * ns)
    assert num_tokens % (nc * ns) == 0 and tpw % chunk == 0 and seqlen % tpw == 0
    nchunks = tpw // chunk
    wpb = seqlen // tpw
    rps = _VPAD // ns

    mesh = plsc.VectorSubcoreMesh(core_axis_name="c", subcore_axis_name="s")

    @functools.partial(
        pl.kernel,
        mesh=mesh,
        out_type=(
            jax.ShapeDtypeStruct((batch, seqlen, 2 * _D), jnp.float32),
            jax.ShapeDtypeStruct((nc, _VPAD, 2 * _D), jnp.float32),
        ),
        scratch_types=[
            pltpu.VMEM((tpw,), jnp.int32),
            pltpu.VMEM((rps,), jnp.int32),
            pltpu.VMEM((_NBUF, chunk, 2 * _D), jnp.float32),
            pltpu.SemaphoreType.DMA,
            pltpu.SemaphoreType.DMA,
        ],
    )
    def lookup(x_hbm, c2a_hbm, codon_hbm, aa_hbm, out_hbm, fused_hbm,
               x_v, idx_v, buf, gsem, wsem):
        sc = lax.axis_index("c")
        sid = lax.axis_index("s")
        wid = sid * nc + sc
        b = wid // wpb
        l0 = (wid % wpb) * tpw
        r0 = sid * rps

        pltpu.sync_copy(x_hbm.at[b, pl.ds(l0, tpw)], x_v)

        myfused = fused_hbm.at[sc]
        row_v = buf.at[0, pl.ds(0, rps), pl.ds(0, _D)]
        pltpu.sync_copy(c2a_hbm.at[pl.ds(r0, rps)], idx_v)
        pltpu.async_copy(aa_hbm.at[idx_v], row_v, gsem).wait()
        pltpu.sync_copy(row_v, myfused.at[pl.ds(r0, rps), pl.ds(_D, _D)])
        pltpu.sync_copy(codon_hbm.at[pl.ds(r0, rps)], row_v)
        pltpu.sync_copy(row_v, myfused.at[pl.ds(r0, rps), pl.ds(0, _D)])
        plsc.subcore_barrier()

        def fire(c, slot):
            return pltpu.async_copy(
                myfused.at[x_v.at[pl.ds(c * chunk, chunk)]], buf.at[slot], gsem)

        def drain(c, slot):
            return pltpu.async_copy(
                buf.at[slot],
                out_hbm.at[b, pl.ds(l0 + c * chunk, chunk)], wsem)

        gs = [fire(0, 0), fire(1, 1)]
        writes = [None] * _NBUF
        for c in range(nchunks):
            slot = c % _NBUF
            gs[c % 2].wait()
            if c + 2 < nchunks:
                nslot = (c + 2) % _NBUF
                if writes[nslot] is not None:
                    writes[nslot].wait()
                gs[c % 2] = fire(c + 2, nslot)
            writes[slot] = drain(c, slot)
        for w in writes:
            if w is not None:
                w.wait()

    return lookup


def kernel(x, aa_table, codon_table, codon_to_aa):
    b, l = x.shape
    v = codon_table.shape[0]
    xi = x.astype(jnp.int32)
    c2a = jnp.pad(codon_to_aa.astype(jnp.int32), (0, _VPAD - v))
    codon_p = jnp.pad(codon_table, ((0, _VPAD - v), (0, 0)))
    lookup = _build_lookup(b, l, 16)
    out, _ = lookup(xi, c2a, codon_p, aa_table)
    return out


# speedup vs baseline: 5.4973x; 1.4243x over previous
import functools

import jax
import jax.numpy as jnp
from jax import lax
from jax.experimental import pallas as pl
from jax.experimental.pallas import tpu as pltpu
from jax.experimental.pallas import tpu_sc as plsc

_D = 1280
_VPAD = 128
_NBUF = 2


def _build_lookup(batch: int, seqlen: int, chunk: int):
    info = plsc.get_sparse_core_info()
    nc, ns = info.num_cores, info.num_subcores
    num_tokens = batch * seqlen
    tpw = num_tokens // (nc * ns)
    assert num_tokens % (nc * ns) == 0 and tpw % chunk == 0 and seqlen % tpw == 0
    nchunks = tpw // chunk
    wpb = seqlen // tpw
    rps = _VPAD // ns

    mesh = plsc.VectorSubcoreMesh(core_axis_name="c", subcore_axis_name="s")

    @functools.partial(
        pl.kernel,
        mesh=mesh,
        out_type=jax.ShapeDtypeStruct((batch, seqlen, 2 * _D), jnp.float32),
        scratch_types=[
            pltpu.VMEM((tpw,), jnp.int32),
            pltpu.VMEM((rps,), jnp.int32),
            pltpu.VMEM((_NBUF, chunk, 2 * _D), jnp.float32),
            pltpu.VMEM_SHARED((_VPAD, 2 * _D), jnp.float32),
            pltpu.SemaphoreType.DMA,
            pltpu.SemaphoreType.DMA,
        ],
    )
    def lookup(x_hbm, c2a_hbm, codon_hbm, aa_hbm, out_hbm,
               x_v, idx_v, buf, fused_sp, gsem, wsem):
        sc = lax.axis_index("c")
        sid = lax.axis_index("s")
        wid = sid * nc + sc
        b = wid // wpb
        l0 = (wid % wpb) * tpw
        r0 = sid * rps

        pltpu.sync_copy(x_hbm.at[b, pl.ds(l0, tpw)], x_v)

        row_v = buf.at[0, pl.ds(0, rps), pl.ds(0, _D)]
        pltpu.sync_copy(c2a_hbm.at[pl.ds(r0, rps)], idx_v)
        pltpu.async_copy(aa_hbm.at[idx_v], row_v, gsem).wait()
        pltpu.sync_copy(row_v, fused_sp.at[pl.ds(r0, rps), pl.ds(_D, _D)])
        pltpu.sync_copy(codon_hbm.at[pl.ds(r0, rps)], row_v)
        pltpu.sync_copy(row_v, fused_sp.at[pl.ds(r0, rps), pl.ds(0, _D)])
        plsc.subcore_barrier()

        def wait_one_write():
            pltpu.make_async_copy(
                buf.at[0], out_hbm.at[b, pl.ds(l0, chunk)], wsem).wait()

        def body(i, _):
            @pl.when(i > 0)
            def _waits():
                wait_one_write()
                wait_one_write()

            for slot in range(_NBUF):
                c = i * _NBUF + slot
                xv = x_v[pl.ds(c * chunk, chunk)]
                rows = []
                for j in range(chunk):
                    rows.append(pltpu.async_copy(
                        fused_sp.at[pl.ds(xv[j], 1)],
                        buf.at[slot, pl.ds(j, 1)], gsem))
                for r in rows:
                    r.wait()
                pltpu.async_copy(
                    buf.at[slot],
                    out_hbm.at[b, pl.ds(l0 + c * chunk, chunk)], wsem)
            return 0

        lax.fori_loop(0, nchunks // _NBUF, body, 0)
        wait_one_write()
        wait_one_write()

    return lookup


def kernel(x, aa_table, codon_table, codon_to_aa):
    b, l = x.shape
    v = codon_table.shape[0]
    xi = x.astype(jnp.int32)
    c2a = jnp.pad(codon_to_aa.astype(jnp.int32), (0, _VPAD - v))
    codon_p = jnp.pad(codon_table, ((0, _VPAD - v), (0, 0)))
    lookup = _build_lookup(b, l, 16)
    return lookup(xi, c2a, codon_p, aa_table)


# speedup vs baseline: 6.8283x; 1.2421x over previous
import functools

import jax
import jax.numpy as jnp
from jax import lax
from jax.experimental import pallas as pl
from jax.experimental.pallas import tpu as pltpu
from jax.experimental.pallas import tpu_sc as plsc

_D = 1280
_VPAD = 72
_RPS = 8
_NBUF = 5


def _build_lookup(batch: int, seqlen: int, chunk: int):
    info = plsc.get_sparse_core_info()
    nc, ns = info.num_cores, info.num_subcores
    num_tokens = batch * seqlen
    tpw = num_tokens // (nc * ns)
    assert num_tokens % (nc * ns) == 0 and tpw % chunk == 0 and seqlen % tpw == 0
    nchunks = tpw // chunk
    wpb = seqlen // tpw
    nbuilders = _VPAD // _RPS

    mesh = plsc.VectorSubcoreMesh(core_axis_name="c", subcore_axis_name="s")

    @functools.partial(
        pl.kernel,
        mesh=mesh,
        out_type=jax.ShapeDtypeStruct((batch, seqlen, 2 * _D), jnp.float32),
        scratch_types=[
            pltpu.VMEM((tpw,), jnp.int32),
            pltpu.VMEM((_RPS,), jnp.int32),
            pltpu.VMEM((_NBUF, chunk, 2 * _D), jnp.float32),
            pltpu.VMEM_SHARED((_VPAD, 2 * _D), jnp.float32),
            pltpu.SemaphoreType.DMA,
            pltpu.SemaphoreType.DMA,
        ],
    )
    def lookup(x_hbm, c2a_hbm, codon_hbm, aa_hbm, out_hbm,
               x_v, idx_v, buf, fused_sp, gsem, wsem):
        sc = lax.axis_index("c")
        sid = lax.axis_index("s")
        wid = sid * nc + sc
        b = wid // wpb
        l0 = (wid % wpb) * tpw

        pltpu.sync_copy(x_hbm.at[b, pl.ds(l0, tpw)], x_v)

        @pl.when(sid < nbuilders)
        def _build():
            r0 = sid * _RPS
            vtail = codon_hbm.shape[0] - (nbuilders - 1) * _RPS
            stage = buf.at[0, pl.ds(0, _RPS), pl.ds(0, _D)]
            pltpu.sync_copy(c2a_hbm.at[pl.ds(r0, _RPS)], idx_v)
            ga = pltpu.async_copy(aa_hbm.at[idx_v], stage, gsem)

            @pl.when(sid < nbuilders - 1)
            def _codon_full():
                pltpu.sync_copy(codon_hbm.at[pl.ds(r0, _RPS)],
                                fused_sp.at[pl.ds(r0, _RPS), pl.ds(0, _D)])

            @pl.when(sid == nbuilders - 1)
            def _codon_tail():
                t0 = (nbuilders - 1) * _RPS
                pltpu.sync_copy(codon_hbm.at[pl.ds(t0, vtail)],
                                fused_sp.at[pl.ds(t0, vtail), pl.ds(0, _D)])

            ga.wait()
            pltpu.sync_copy(stage,
                            fused_sp.at[pl.ds(r0, _RPS), pl.ds(_D, _D)])
        plsc.subcore_barrier()

        def wait_one_write():
            pltpu.make_async_copy(
                buf.at[0], out_hbm.at[b, pl.ds(l0, chunk)], wsem).wait()

        def body(c, _):
            slot = lax.rem(c, _NBUF)

            @pl.when(c >= _NBUF)
            def _wait():
                wait_one_write()

            xv = x_v[pl.ds(c * chunk, chunk)]
            rows = []
            for j in range(chunk):
                rows.append(pltpu.async_copy(
                    fused_sp.at[pl.ds(xv[j], 1)],
                    buf.at[slot, pl.ds(j, 1)], gsem))
            for r in rows:
                r.wait()
            pltpu.async_copy(
                buf.at[slot],
                out_hbm.at[b, pl.ds(l0 + c * chunk, chunk)], wsem)
            return 0

        lax.fori_loop(0, nchunks, body, 0)
        for _ in range(_NBUF):
            wait_one_write()

    return lookup


def kernel(x, aa_table, codon_table, codon_to_aa):
    b, l = x.shape
    v = codon_table.shape[0]
    xi = x.astype(jnp.int32)
    c2a = jnp.pad(codon_to_aa.astype(jnp.int32), (0, _VPAD - v))
    lookup = _build_lookup(b, l, 8)
    return lookup(xi, c2a, codon_table, aa_table)


# speedup vs baseline: 6.9157x; 1.0128x over previous
import functools

import jax
import jax.numpy as jnp
from jax import lax
from jax.experimental import pallas as pl
from jax.experimental.pallas import tpu as pltpu
from jax.experimental.pallas import tpu_sc as plsc

_D = 1280
_VPAD = 72
_RPS = 8
_NBUF = 2


def _build_lookup(batch: int, seqlen: int, chunk: int):
    info = plsc.get_sparse_core_info()
    nc, ns = info.num_cores, info.num_subcores
    num_tokens = batch * seqlen
    tpw = num_tokens // (nc * ns)
    assert num_tokens % (nc * ns) == 0 and tpw % chunk == 0 and seqlen % tpw == 0
    nchunks = tpw // chunk
    wpb = seqlen // tpw
    nbuilders = _VPAD // _RPS

    mesh = plsc.VectorSubcoreMesh(core_axis_name="c", subcore_axis_name="s")

    @functools.partial(
        pl.kernel,
        mesh=mesh,
        out_type=jax.ShapeDtypeStruct((batch, seqlen, 2 * _D), jnp.float32),
        scratch_types=[
            pltpu.VMEM((tpw,), jnp.int32),
            pltpu.VMEM((16,), jnp.int32),
            pltpu.VMEM((_RPS, _D), jnp.float32),
            pltpu.VMEM((_NBUF, chunk, 2 * _D), jnp.float32),
            pltpu.VMEM_SHARED((_VPAD, 2 * _D), jnp.float32),
            pltpu.SemaphoreType.DMA,
            pltpu.SemaphoreType.DMA,
        ],
    )
    def lookup(x_hbm, c2a_hbm, codon_hbm, aa_hbm, out_hbm,
               x_v, idx_v, stage_v, buf, fused_sp, gsem, wsem):
        sc = lax.axis_index("c")
        sid = lax.axis_index("s")
        wid = sid * nc + sc
        b = wid // wpb
        l0 = (wid % wpb) * tpw

        pltpu.sync_copy(x_hbm.at[b, pl.ds(l0, tpw)], x_v)

        t0 = (nbuilders - 1) * _RPS
        vtail = codon_hbm.shape[0] - t0

        @pl.when(sid < nbuilders - 1)
        def _build():
            r0 = sid * _RPS
            pltpu.sync_copy(c2a_hbm.at[pl.ds(r0, _RPS)], idx_v.at[pl.ds(0, _RPS)])
            ga = pltpu.async_copy(
                aa_hbm.at[idx_v.at[pl.ds(0, _RPS)]], stage_v, gsem)
            pltpu.sync_copy(codon_hbm.at[pl.ds(r0, _RPS)],
                            fused_sp.at[pl.ds(r0, _RPS), pl.ds(0, _D)])
            ga.wait()
            pltpu.sync_copy(stage_v,
                            fused_sp.at[pl.ds(r0, _RPS), pl.ds(_D, _D)])

        @pl.when(sid == nbuilders - 1)
        def _build_tail():
            idx_v[...] = jnp.zeros((16,), jnp.int32)
            pltpu.sync_copy(c2a_hbm.at[pl.ds(t0, vtail)],
                            idx_v.at[pl.ds(0, vtail)])
            ga = pltpu.async_copy(
                aa_hbm.at[idx_v.at[pl.ds(0, _RPS)]], stage_v, gsem)
            pltpu.sync_copy(codon_hbm.at[pl.ds(t0, vtail)],
                            fused_sp.at[pl.ds(t0, vtail), pl.ds(0, _D)])
            ga.wait()
            pltpu.sync_copy(stage_v,
                            fused_sp.at[pl.ds(t0, _RPS), pl.ds(_D, _D)])
        plsc.subcore_barrier()

        def wait_one_write():
            pltpu.make_async_copy(
                buf.at[0], out_hbm.at[b, pl.ds(l0, chunk)], wsem).wait()

        def body(c, _):
            slot = lax.rem(c, _NBUF)

            @pl.when(c >= _NBUF)
            def _wait():
                wait_one_write()

            xv = x_v[pl.ds(c * chunk, chunk)]
            rows = []
            for j in range(chunk):
                rows.append(pltpu.async_copy(
                    fused_sp.at[pl.ds(xv[j], 1)],
                    buf.at[slot, pl.ds(j, 1)], gsem))
            for r in rows:
                r.wait()
            pltpu.async_copy(
                buf.at[slot],
                out_hbm.at[b, pl.ds(l0 + c * chunk, chunk)], wsem)
            return 0

        lax.fori_loop(0, nchunks, body, 0)
        for _ in range(_NBUF):
            wait_one_write()

    return lookup


def kernel(x, aa_table, codon_table, codon_to_aa):
    b, l = x.shape
    v = codon_table.shape[0]
    xi = x.astype(jnp.int32)
    c2a = codon_to_aa.astype(jnp.int32)
    lookup = _build_lookup(b, l, 16)
    return lookup(xi, c2a, codon_table, aa_table)
